# Initial kernel scaffold; baseline (speedup 1.0000x reference)
import jax
import jax.numpy as jnp
from jax.experimental import pallas as pl


def kernel(e, index, W, b):
    raise NotImplementedError("write your pallas kernel here")


# speedup vs baseline: 2.0747x; 2.0747x over previous
import jax
import jax.numpy as jnp
from jax.experimental import pallas as pl
from jax.experimental.pallas import tpu as pltpu

N_OUT = 10000
R = 512
WS = 512
ACC_ROWS = ((N_OUT - 1) // WS) * WS + 2 * WS


def _body(e_ref, idx_ref, w_ref, b_ref, out_ref, acc_ref):
    i = pl.program_id(0)

    @pl.when(i == 0)
    def _init():
        acc_ref[...] = jnp.zeros_like(acc_ref)

    h = jax.lax.dot_general(
        e_ref[...], w_ref[...], (((1,), (1,)), ((), ())),
        preferred_element_type=jnp.float32)
    h = jnp.maximum(h + b_ref[...], 0.0)

    idx = idx_ref[0, 0, :]
    lo = jnp.min(idx)
    hi = jnp.max(idx)
    base0 = (lo // WS) * WS
    nwin = (hi - base0) // WS + 1

    def wbody(w, carry):
        base = base0 + w * WS
        local = idx - base
        oh = (local[:, None]
              == jax.lax.broadcasted_iota(jnp.int32, (R, WS), 1)
              ).astype(jnp.float32)
        part = jax.lax.dot_general(
            oh, h, (((0,), (0,)), ((), ())),
            preferred_element_type=jnp.float32)
        acc_ref[pl.ds(base, WS), :] += part
        return carry

    jax.lax.fori_loop(0, nwin, wbody, 0)

    @pl.when(i == pl.num_programs(0) - 1)
    def _flush():
        out_ref[...] = acc_ref[:N_OUT, :]


def kernel(e, index, W, b):
    n_e, d = e.shape
    nb = n_e // R
    idx3 = index.reshape(nb, 1, R)
    b2 = b.reshape(1, d)
    return pl.pallas_call(
        _body,
        grid=(nb,),
        in_specs=[
            pl.BlockSpec((R, d), lambda i: (i, 0)),
            pl.BlockSpec((1, 1, R), lambda i: (i, 0, 0)),
            pl.BlockSpec((d, d), lambda i: (0, 0)),
            pl.BlockSpec((1, d), lambda i: (0, 0)),
        ],
        out_specs=pl.BlockSpec((N_OUT, d), lambda i: (0, 0)),
        out_shape=jax.ShapeDtypeStruct((N_OUT, d), jnp.float32),
        scratch_shapes=[pltpu.VMEM((ACC_ROWS, d), jnp.float32)],
    )(e, idx3, W, b2)


# speedup vs baseline: 2.1364x; 1.0298x over previous
import jax
import jax.numpy as jnp
from jax.experimental import pallas as pl
from jax.experimental.pallas import tpu as pltpu

N_OUT = 10000
R = 512
WS = 128
ACC_ROWS = ((N_OUT - 1) // WS) * WS + 2 * WS


def _body(e_ref, idx_ref, w_ref, b_ref, out_ref, acc_ref):
    i = pl.program_id(0)

    @pl.when(i == 0)
    def _init():
        acc_ref[...] = jnp.zeros_like(acc_ref)

    h = jax.lax.dot_general(
        e_ref[...], w_ref[...], (((1,), (1,)), ((), ())),
        preferred_element_type=jnp.float32)
    h = jnp.maximum(h + b_ref[...], 0.0)

    h16 = h.astype(jnp.bfloat16)
    idx = idx_ref[0, 0, :]
    lo = jnp.min(idx)
    hi = jnp.max(idx)
    base0 = (lo // WS) * WS
    nwin = (hi - base0) // WS + 1

    def wbody(w, carry):
        base = base0 + w * WS
        local = idx - base
        oh = (local[:, None]
              == jax.lax.broadcasted_iota(jnp.int32, (R, WS), 1)
              ).astype(jnp.bfloat16)
        part = jax.lax.dot_general(
            oh, h16, (((0,), (0,)), ((), ())),
            preferred_element_type=jnp.float32)
        acc_ref[pl.ds(base, WS), :] += part
        return carry

    jax.lax.fori_loop(0, nwin, wbody, 0)

    @pl.when(i == pl.num_programs(0) - 1)
    def _flush():
        out_ref[...] = acc_ref[:N_OUT, :]


def kernel(e, index, W, b):
    n_e, d = e.shape
    nb = n_e // R
    idx3 = index.reshape(nb, 1, R)
    b2 = b.reshape(1, d)
    return pl.pallas_call(
        _body,
        grid=(nb,),
        in_specs=[
            pl.BlockSpec((R, d), lambda i: (i, 0)),
            pl.BlockSpec((1, 1, R), lambda i: (i, 0, 0)),
            pl.BlockSpec((d, d), lambda i: (0, 0)),
            pl.BlockSpec((1, d), lambda i: (0, 0)),
        ],
        out_specs=pl.BlockSpec((N_OUT, d), lambda i: (0, 0)),
        out_shape=jax.ShapeDtypeStruct((N_OUT, d), jnp.float32),
        scratch_shapes=[pltpu.VMEM((ACC_ROWS, d), jnp.float32)],
    )(e, idx3, W, b2)


# speedup vs baseline: 2.4775x; 1.1597x over previous
import jax
import jax.numpy as jnp
from jax.experimental import pallas as pl
from jax.experimental.pallas import tpu as pltpu

N_OUT = 10000
R = 512
WS = 128
ACC_ROWS = ((N_OUT - 1) // WS) * WS + 2 * WS


def _body(base0s_ref, nwins_ref, e_ref, idx_ref, w_ref, b_ref, out_ref,
          acc_ref):
    i = pl.program_id(0)

    @pl.when(i == 0)
    def _init():
        acc_ref[...] = jnp.zeros_like(acc_ref)

    h = jax.lax.dot_general(
        e_ref[...].astype(jnp.bfloat16), w_ref[...],
        (((1,), (1,)), ((), ())),
        preferred_element_type=jnp.float32)
    h16 = jnp.maximum(h + b_ref[...], 0.0).astype(jnp.bfloat16)

    idx = idx_ref[0, :, :]
    base0 = base0s_ref[i]
    nwin = nwins_ref[i]

    def wbody(w, carry):
        base = base0 + w * WS
        seg = jax.lax.broadcasted_iota(jnp.int32, (WS, R), 0) + base
        oht = (seg == idx).astype(jnp.bfloat16)
        part = jax.lax.dot_general(
            oht, h16, (((1,), (0,)), ((), ())),
            preferred_element_type=jnp.float32)
        acc_ref[pl.ds(base, WS), :] += part
        return carry

    jax.lax.fori_loop(0, nwin, wbody, 0)

    @pl.when(i == pl.num_programs(0) - 1)
    def _flush():
        out_ref[...] = acc_ref[:N_OUT, :]


def kernel(e, index, W, b):
    n_e, d = e.shape
    nb = n_e // R
    idx3 = index.reshape(nb, 1, R)
    b2 = b.reshape(1, d)
    lo = index[::R]
    hi = index[R - 1::R]
    base0s = (lo // WS) * WS
    nwins = (hi - base0s) // WS + 1
    grid_spec = pltpu.PrefetchScalarGridSpec(
        num_scalar_prefetch=2,
        grid=(nb,),
        in_specs=[
            pl.BlockSpec((R, d), lambda i, *_: (i, 0)),
            pl.BlockSpec((1, 1, R), lambda i, *_: (i, 0, 0)),
            pl.BlockSpec((d, d), lambda i, *_: (0, 0)),
            pl.BlockSpec((1, d), lambda i, *_: (0, 0)),
        ],
        out_specs=pl.BlockSpec((N_OUT, d), lambda i, *_: (0, 0)),
        scratch_shapes=[pltpu.VMEM((ACC_ROWS, d), jnp.float32)],
    )
    return pl.pallas_call(
        _body,
        grid_spec=grid_spec,
        out_shape=jax.ShapeDtypeStruct((N_OUT, d), jnp.float32),
    )(base0s, nwins, e, idx3, W.astype(jnp.bfloat16), b2)


# speedup vs baseline: 4.6809x; 1.8894x over previous
import jax
import jax.numpy as jnp
from jax.experimental import pallas as pl
from jax.experimental.pallas import tpu as pltpu

N_OUT = 10000
R = 1280
WS = 256
ACC_ROWS = ((N_OUT - 1) // WS) * WS + 2 * WS


def _body(base0s_ref, nwins_ref, e_ref, idx_ref, w_ref, b_ref, out_ref,
          acc_ref):
    i = pl.program_id(0)

    @pl.when(i == 0)
    def _init():
        acc_ref[...] = jnp.zeros_like(acc_ref)

    h = jax.lax.dot_general(
        e_ref[...].astype(jnp.bfloat16), w_ref[...],
        (((1,), (1,)), ((), ())),
        preferred_element_type=jnp.float32)
    h16 = jnp.maximum(h + b_ref[...], 0.0).astype(jnp.bfloat16)

    idx = idx_ref[0, :, :]
    base0 = base0s_ref[i]
    nwin = nwins_ref[i]

    def wbody(w, carry):
        base = base0 + w * WS
        seg = jax.lax.broadcasted_iota(jnp.int32, (WS, R), 0) + base
        oht = (seg == idx).astype(jnp.bfloat16)
        part = jax.lax.dot_general(
            oht, h16, (((1,), (0,)), ((), ())),
            preferred_element_type=jnp.float32)
        acc_ref[pl.ds(base, WS), :] += part
        return carry

    jax.lax.fori_loop(0, nwin, wbody, 0)

    @pl.when(i == pl.num_programs(0) - 1)
    def _flush():
        out_ref[...] = acc_ref[:N_OUT, :]


def kernel(e, index, W, b):
    n_e, d = e.shape
    nb = n_e // R
    idx3 = index.reshape(nb, 1, R)
    b2 = b.reshape(1, d)
    lo = index[::R]
    hi = index[R - 1::R]
    base0s = (lo // WS) * WS
    nwins = (hi - base0s) // WS + 1
    grid_spec = pltpu.PrefetchScalarGridSpec(
        num_scalar_prefetch=2,
        grid=(nb,),
        in_specs=[
            pl.BlockSpec((R, d), lambda i, *_: (i, 0)),
            pl.BlockSpec((1, 1, R), lambda i, *_: (i, 0, 0)),
            pl.BlockSpec((d, d), lambda i, *_: (0, 0)),
            pl.BlockSpec((1, d), lambda i, *_: (0, 0)),
        ],
        out_specs=pl.BlockSpec((N_OUT, d), lambda i, *_: (0, 0)),
        scratch_shapes=[pltpu.VMEM((ACC_ROWS, d), jnp.float32)],
    )
    return pl.pallas_call(
        _body,
        grid_spec=grid_spec,
        out_shape=jax.ShapeDtypeStruct((N_OUT, d), jnp.float32),
    )(base0s, nwins, e, idx3, W.astype(jnp.bfloat16), b2)


# speedup vs baseline: 6.7027x; 1.4319x over previous
import jax
import jax.numpy as jnp
from jax.experimental import pallas as pl
from jax.experimental.pallas import tpu as pltpu

N_OUT = 10000
R = 2560
WS = 256
ACC_ROWS = ((N_OUT - 1) // WS) * WS + 2 * WS


def _body(base0s_ref, nwins_ref, e_ref, idx_ref, w_ref, b_ref, out_ref,
          acc_ref):
    i = pl.program_id(0)

    @pl.when(i == 0)
    def _init():
        acc_ref[...] = jnp.zeros_like(acc_ref)

    h = jax.lax.dot_general(
        e_ref[...].astype(jnp.bfloat16), w_ref[...],
        (((1,), (1,)), ((), ())),
        preferred_element_type=jnp.float32)
    h16 = jnp.maximum(h + b_ref[...], 0.0).astype(jnp.bfloat16)

    idx = idx_ref[0, :, :]
    base0 = base0s_ref[i]
    nwin = nwins_ref[i]

    def wbody(w, carry):
        base = base0 + w * WS
        seg = jax.lax.broadcasted_iota(jnp.int32, (WS, R), 0) + base
        oht = (seg == idx).astype(jnp.bfloat16)
        part = jax.lax.dot_general(
            oht, h16, (((1,), (0,)), ((), ())),
            preferred_element_type=jnp.float32)
        acc_ref[pl.ds(base, WS), :] += part
        return carry

    jax.lax.fori_loop(0, nwin, wbody, 0)

    @pl.when(i == pl.num_programs(0) - 1)
    def _flush():
        out_ref[...] = acc_ref[:N_OUT, :]


def kernel(e, index, W, b):
    n_e, d = e.shape
    nb = n_e // R
    idx3 = index.reshape(nb, 1, R)
    b2 = b.reshape(1, d)
    lo = index[::R]
    hi = index[R - 1::R]
    base0s = (lo // WS) * WS
    nwins = (hi - base0s) // WS + 1
    grid_spec = pltpu.PrefetchScalarGridSpec(
        num_scalar_prefetch=2,
        grid=(nb,),
        in_specs=[
            pl.BlockSpec((R, d), lambda i, *_: (i, 0)),
            pl.BlockSpec((1, 1, R), lambda i, *_: (i, 0, 0)),
            pl.BlockSpec((d, d), lambda i, *_: (0, 0)),
            pl.BlockSpec((1, d), lambda i, *_: (0, 0)),
        ],
        out_specs=pl.BlockSpec((N_OUT, d), lambda i, *_: (0, 0)),
        scratch_shapes=[pltpu.VMEM((ACC_ROWS, d), jnp.float32)],
    )
    return pl.pallas_call(
        _body,
        grid_spec=grid_spec,
        out_shape=jax.ShapeDtypeStruct((N_OUT, d), jnp.float32),
    )(base0s, nwins, e, idx3, W.astype(jnp.bfloat16), b2)


# speedup vs baseline: 7.5062x; 1.1199x over previous
import jax
import jax.numpy as jnp
from jax.experimental import pallas as pl
from jax.experimental.pallas import tpu as pltpu

N_OUT = 10000
R = 4000
WS = 256
ACC_ROWS = ((N_OUT - 1) // WS) * WS + 2 * WS


def _body(base0s_ref, nwins_ref, e_ref, idx_ref, w_ref, b_ref, out_ref,
          acc_ref):
    i = pl.program_id(0)

    @pl.when(i == 0)
    def _init():
        acc_ref[...] = jnp.zeros_like(acc_ref)

    h = jax.lax.dot_general(
        e_ref[...].astype(jnp.bfloat16), w_ref[...],
        (((1,), (1,)), ((), ())),
        preferred_element_type=jnp.float32)
    h16 = jnp.maximum(h + b_ref[...], 0.0).astype(jnp.bfloat16)

    idx = idx_ref[0, :, :]
    base0 = base0s_ref[i]
    nwin = nwins_ref[i]

    def wbody(w, carry):
        base = base0 + w * WS
        seg = jax.lax.broadcasted_iota(jnp.int32, (WS, R), 0) + base
        oht = (seg == idx).astype(jnp.bfloat16)
        part = jax.lax.dot_general(
            oht, h16, (((1,), (0,)), ((), ())),
            preferred_element_type=jnp.float32)
        acc_ref[pl.ds(base, WS), :] += part
        return carry

    jax.lax.fori_loop(0, nwin, wbody, 0)

    @pl.when(i == pl.num_programs(0) - 1)
    def _flush():
        out_ref[...] = acc_ref[:N_OUT, :]


def kernel(e, index, W, b):
    n_e, d = e.shape
    nb = n_e // R
    idx3 = index.reshape(nb, 1, R)
    b2 = b.reshape(1, d)
    lo = index[::R]
    hi = index[R - 1::R]
    base0s = (lo // WS) * WS
    nwins = (hi - base0s) // WS + 1
    grid_spec = pltpu.PrefetchScalarGridSpec(
        num_scalar_prefetch=2,
        grid=(nb,),
        in_specs=[
            pl.BlockSpec((R, d), lambda i, *_: (i, 0)),
            pl.BlockSpec((1, 1, R), lambda i, *_: (i, 0, 0)),
            pl.BlockSpec((d, d), lambda i, *_: (0, 0)),
            pl.BlockSpec((1, d), lambda i, *_: (0, 0)),
        ],
        out_specs=pl.BlockSpec((N_OUT, d), lambda i, *_: (0, 0)),
        scratch_shapes=[pltpu.VMEM((ACC_ROWS, d), jnp.float32)],
    )
    return pl.pallas_call(
        _body,
        grid_spec=grid_spec,
        out_shape=jax.ShapeDtypeStruct((N_OUT, d), jnp.float32),
    )(base0s, nwins, e, idx3, W.astype(jnp.bfloat16), b2)


# speedup vs baseline: 7.9489x; 1.0590x over previous
import jax
import jax.numpy as jnp
from jax.experimental import pallas as pl
from jax.experimental.pallas import tpu as pltpu

N_OUT = 10000
R = 8000
WS = 256
ACC_ROWS = ((N_OUT - 1) // WS) * WS + 2 * WS


def _body(base0s_ref, nwins_ref, e_ref, idx_ref, w_ref, b_ref, out_ref,
          acc_ref):
    i = pl.program_id(0)

    @pl.when(i == 0)
    def _init():
        acc_ref[...] = jnp.zeros_like(acc_ref)

    h = jax.lax.dot_general(
        e_ref[...].astype(jnp.bfloat16), w_ref[...],
        (((1,), (1,)), ((), ())),
        preferred_element_type=jnp.float32)
    h16 = jnp.maximum(h + b_ref[...], 0.0).astype(jnp.bfloat16)

    idx = idx_ref[0, :, :]
    base0 = base0s_ref[i]
    nwin = nwins_ref[i]

    def wbody(w, carry):
        base = base0 + w * WS
        seg = jax.lax.broadcasted_iota(jnp.int32, (WS, R), 0) + base
        oht = (seg == idx).astype(jnp.bfloat16)
        part = jax.lax.dot_general(
            oht, h16, (((1,), (0,)), ((), ())),
            preferred_element_type=jnp.float32)
        acc_ref[pl.ds(base, WS), :] += part
        return carry

    jax.lax.fori_loop(0, nwin, wbody, 0)

    @pl.when(i == pl.num_programs(0) - 1)
    def _flush():
        out_ref[...] = acc_ref[:N_OUT, :]


def kernel(e, index, W, b):
    n_e, d = e.shape
    nb = n_e // R
    idx3 = index.reshape(nb, 1, R)
    b2 = b.reshape(1, d)
    lo = index[::R]
    hi = index[R - 1::R]
    base0s = (lo // WS) * WS
    nwins = (hi - base0s) // WS + 1
    grid_spec = pltpu.PrefetchScalarGridSpec(
        num_scalar_prefetch=2,
        grid=(nb,),
        in_specs=[
            pl.BlockSpec((R, d), lambda i, *_: (i, 0)),
            pl.BlockSpec((1, 1, R), lambda i, *_: (i, 0, 0)),
            pl.BlockSpec((d, d), lambda i, *_: (0, 0)),
            pl.BlockSpec((1, d), lambda i, *_: (0, 0)),
        ],
        out_specs=pl.BlockSpec((N_OUT, d), lambda i, *_: (0, 0)),
        scratch_shapes=[pltpu.VMEM((ACC_ROWS, d), jnp.float32)],
    )
    return pl.pallas_call(
        _body,
        grid_spec=grid_spec,
        out_shape=jax.ShapeDtypeStruct((N_OUT, d), jnp.float32),
    )(base0s, nwins, e, idx3, W.astype(jnp.bfloat16), b2)
